# Initial kernel scaffold; baseline (speedup 1.0000x reference)
import functools

import jax
import jax.numpy as jnp
from jax import lax
from jax.experimental import pallas as pl
from jax.experimental.pallas import tpu as pltpu
from jax.experimental.pallas import tpu_sc as plsc

N = 10000
E = 320000
D = 128
L = 16
NC = 2
NS = 16
NW = NC * NS
EPT = E // NW
K = 80
NCHUNK = EPT // K
RPT = N // NS
ZROWS = 125


def _spmm_body(rows_hbm, cols_hbm, vals_hbm, x_hbm, out_hbm,
               rows_v, cols_v, vals_v, gbuf, zbuf, acc, sem):
    c = lax.axis_index("c")
    s = lax.axis_index("s")
    wid = s * NC + c

    pltpu.sync_copy(rows_hbm.at[wid], rows_v)
    pltpu.sync_copy(cols_hbm.at[wid], cols_v)
    pltpu.sync_copy(vals_hbm.at[wid], vals_v)

    def zfill(i, carry):
        for j in range(D // L):
            zbuf[i, pl.ds(j * L, L)] = jnp.zeros((L,), jnp.float32)
        return carry
    lax.fori_loop(0, ZROWS, zfill, 0)
    row0 = s * RPT
    for i in range(RPT // ZROWS):
        pltpu.sync_copy(zbuf, acc.at[pl.ds(row0 + i * ZROWS, ZROWS)])
    plsc.subcore_barrier()

    def chunk_body(ci, carry):
        pltpu.async_copy(x_hbm.at[cols_v.at[ci]], gbuf, sem).wait()

        def edge_body(e, inner):
            v = vals_v[ci, e]
            vv = jnp.full((L,), v, jnp.float32)
            for j in range(D // L):
                sl = pl.ds(j * L, L)
                gbuf[e, sl] = gbuf[e, sl] * vv
            return inner
        lax.fori_loop(0, K, edge_body, 0)

        pltpu.sync_copy(gbuf, acc.at[rows_v.at[ci]], add=True)
        return carry
    lax.fori_loop(0, NCHUNK, chunk_body, 0)

    plsc.subcore_barrier()
    for i in range(RPT // ZROWS):
        r = row0 + i * ZROWS
        pltpu.sync_copy(acc.at[pl.ds(r, ZROWS)], out_hbm.at[c, pl.ds(r, ZROWS)])


_spmm = pl.kernel(
    _spmm_body,
    out_type=jax.ShapeDtypeStruct((NC, N, D), jnp.float32),
    mesh=plsc.VectorSubcoreMesh(core_axis_name="c", subcore_axis_name="s",
                                num_cores=NC, num_subcores=NS),
    scratch_types=[
        pltpu.VMEM((NCHUNK, K), jnp.int32),
        pltpu.VMEM((NCHUNK, K), jnp.int32),
        pltpu.VMEM((NCHUNK, K), jnp.float32),
        pltpu.VMEM((K, D), jnp.float32),
        pltpu.VMEM((ZROWS, D), jnp.float32),
        pltpu.VMEM_SHARED((N, D), jnp.float32),
        pltpu.SemaphoreType.DMA,
    ],
)


def _mm_body(y_ref, w_ref, o_ref):
    o_ref[...] = jnp.dot(y_ref[0] + y_ref[1], w_ref[...],
                         preferred_element_type=jnp.float32)


BM = 1000


def _matmul(y, w):
    return pl.pallas_call(
        _mm_body,
        grid=(N // BM,),
        in_specs=[
            pl.BlockSpec((NC, BM, D), lambda i: (0, i, 0)),
            pl.BlockSpec((D, D), lambda i: (0, 0)),
        ],
        out_specs=pl.BlockSpec((BM, D), lambda i: (i, 0)),
        out_shape=jax.ShapeDtypeStruct((N, D), jnp.float32),
    )(y, w)


def kernel(adj_indices, adj_values, embeds, W):
    rows = adj_indices[0].astype(jnp.int32).reshape(NW, NCHUNK, K)
    cols = adj_indices[1].astype(jnp.int32).reshape(NW, NCHUNK, K)
    vals = adj_values.reshape(NW, NCHUNK, K)
    y = _spmm(rows, cols, vals, embeds)
    return _matmul(y, W)


# speedup vs baseline: 2.9964x; 2.9964x over previous
import jax
import jax.numpy as jnp
from jax import lax
from jax.experimental import pallas as pl
from jax.experimental.pallas import tpu as pltpu
from jax.experimental.pallas import tpu_sc as plsc

N = 10000
E = 320000
D = 128
DH = D // 2
L = 16
NC = 2
NS = 16
NW = NC * NS
EPT = E // NW
K = 80
NCHUNK = EPT // K
NP = 10240
RPT = NP // NS
ZROWS = 128


def _spmm_body(rows_hbm, cols_hbm, vals_hbm, x2_hbm, out_hbm,
               rows_v, cols_v, vals_v, cidx_v, gbuf, zbuf, acc, sem):
    c = lax.axis_index("c")
    s = lax.axis_index("s")
    wid = s * NC + c
    row0 = s * RPT

    pltpu.sync_copy(rows_hbm.at[wid], rows_v)
    pltpu.sync_copy(cols_hbm.at[wid], cols_v)
    pltpu.sync_copy(vals_hbm.at[wid], vals_v)

    def zfill(i, carry):
        for j in range(DH // L):
            zbuf[i, pl.ds(j * L, L)] = jnp.zeros((L,), jnp.float32)
        return carry
    lax.fori_loop(0, ZROWS, zfill, 0)

    for h in range(2):
        for i in range(RPT // ZROWS):
            pltpu.sync_copy(zbuf, acc.at[pl.ds(row0 + i * ZROWS, ZROWS)])
        plsc.subcore_barrier()

        def chunk_body(ci, carry):
            for g in range(K // L):
                sl = pl.ds(g * L, L)
                cidx_v[sl] = cols_v[ci, sl] * 2 + h
            pltpu.async_copy(x2_hbm.at[cidx_v], gbuf, sem).wait()

            def group_body(g, inner):
                vv16 = vals_v[ci, pl.ds(g * L, L)]
                for e in range(L):
                    vv = jnp.full((L,), vv16[e], jnp.float32)
                    row = g * L + e
                    for j in range(DH // L):
                        sl = pl.ds(j * L, L)
                        gbuf[row, sl] = gbuf[row, sl] * vv
                return inner
            lax.fori_loop(0, K // L, group_body, 0)

            pltpu.sync_copy(gbuf, acc.at[rows_v.at[ci]], add=True)
            return carry
        lax.fori_loop(0, NCHUNK, chunk_body, 0)

        plsc.subcore_barrier()
        for i in range(RPT // ZROWS):
            r = row0 + i * ZROWS
            pltpu.sync_copy(acc.at[pl.ds(r, ZROWS)],
                            out_hbm.at[c, h, pl.ds(r, ZROWS)])


_spmm = pl.kernel(
    _spmm_body,
    out_type=jax.ShapeDtypeStruct((NC, 2, NP, DH), jnp.float32),
    mesh=plsc.VectorSubcoreMesh(core_axis_name="c", subcore_axis_name="s",
                                num_cores=NC, num_subcores=NS),
    scratch_types=[
        pltpu.VMEM((NCHUNK, K), jnp.int32),
        pltpu.VMEM((NCHUNK, K), jnp.int32),
        pltpu.VMEM((NCHUNK, K), jnp.float32),
        pltpu.VMEM((K,), jnp.int32),
        pltpu.VMEM((K, DH), jnp.float32),
        pltpu.VMEM((ZROWS, DH), jnp.float32),
        pltpu.VMEM_SHARED((NP, DH), jnp.float32),
        pltpu.SemaphoreType.DMA,
    ],
    compiler_params=pltpu.CompilerParams(use_tc_tiling_on_sc=False),
)


def _mm_body(y_ref, w_ref, o_ref):
    o_ref[...] = (
        jnp.dot(y_ref[0, 0] + y_ref[1, 0], w_ref[0],
                preferred_element_type=jnp.float32)
        + jnp.dot(y_ref[0, 1] + y_ref[1, 1], w_ref[1],
                  preferred_element_type=jnp.float32)
    )


BM = 1000


def _matmul(y, w2):
    return pl.pallas_call(
        _mm_body,
        grid=(N // BM,),
        in_specs=[
            pl.BlockSpec((NC, 2, BM, DH), lambda i: (0, 0, i, 0)),
            pl.BlockSpec((2, DH, D), lambda i: (0, 0, 0)),
        ],
        out_specs=pl.BlockSpec((BM, D), lambda i: (i, 0)),
        out_shape=jax.ShapeDtypeStruct((N, D), jnp.float32),
    )(y, w2)


def kernel(adj_indices, adj_values, embeds, W):
    rows = adj_indices[0].astype(jnp.int32).reshape(NW, NCHUNK, K)
    cols = adj_indices[1].astype(jnp.int32).reshape(NW, NCHUNK, K)
    vals = adj_values.reshape(NW, NCHUNK, K)
    x2 = embeds.reshape(2 * N, DH)
    y = _spmm(rows, cols, vals, x2)
    return _matmul(y[:, :, :N], W.reshape(2, DH, D))


# speedup vs baseline: 4.1694x; 1.3915x over previous
import jax
import jax.numpy as jnp
from jax import lax
from jax.experimental import pallas as pl
from jax.experimental.pallas import tpu as pltpu
from jax.experimental.pallas import tpu_sc as plsc

N = 10000
E = 320000
D = 128
DH = D // 2
L = 16
NC = 2
NS = 16
NW = NC * NS
EPT = E // NW
K = 80
NCHUNK = EPT // K
NP = 10240
RPT = NP // NS
ZROWS = 128


def _spmm_body(rows_hbm, cols2_hbm, vals_hbm, x2_hbm, out_hbm,
               rows_v, cols2_v, vals_v, gbuf, zbuf, acc, gsem0, gsem1):
    c = lax.axis_index("c")
    s = lax.axis_index("s")
    wid = s * NC + c
    row0 = s * RPT

    pltpu.sync_copy(rows_hbm.at[wid], rows_v)
    pltpu.sync_copy(cols2_hbm.at[wid], cols2_v)
    pltpu.sync_copy(vals_hbm.at[wid], vals_v)

    def zfill(i, carry):
        for j in range(DH // L):
            zbuf[i, pl.ds(j * L, L)] = jnp.zeros((L,), jnp.float32)
        return carry
    lax.fori_loop(0, ZROWS, zfill, 0)

    sems = (gsem0, gsem1)

    for h in range(2):
        for i in range(RPT // ZROWS):
            pltpu.sync_copy(zbuf, acc.at[pl.ds(row0 + i * ZROWS, ZROWS)])
        plsc.subcore_barrier()

        def scale_scatter(ci, b):
            pltpu.make_async_copy(
                x2_hbm.at[cols2_v.at[h, 0]], gbuf.at[b], sems[b]).wait()

            def group_body(g, inner):
                vv16 = vals_v[ci, pl.ds(g * L, L)]
                for e in range(L):
                    vv = jnp.full((L,), vv16[e], jnp.float32)
                    row = g * L + e
                    for j in range(DH // L):
                        sl = pl.ds(j * L, L)
                        gbuf[b, row, sl] = gbuf[b, row, sl] * vv
                return inner
            lax.fori_loop(0, K // L, group_body, 0)

            pltpu.sync_copy(gbuf.at[b], acc.at[rows_v.at[ci]], add=True)

        pltpu.async_copy(x2_hbm.at[cols2_v.at[h, 0]], gbuf.at[0], gsem0)

        def pair_body(p, carry):
            ci = 2 * p
            pltpu.async_copy(
                x2_hbm.at[cols2_v.at[h, ci + 1]], gbuf.at[1], gsem1)
            scale_scatter(ci, 0)
            pltpu.async_copy(
                x2_hbm.at[cols2_v.at[h, ci + 2]], gbuf.at[0], gsem0)
            scale_scatter(ci + 1, 1)
            return carry
        lax.fori_loop(0, (NCHUNK - 1) // 2, pair_body, 0)
        scale_scatter(NCHUNK - 1, 0)

        plsc.subcore_barrier()
        for i in range(RPT // ZROWS):
            r = row0 + i * ZROWS
            pltpu.sync_copy(acc.at[pl.ds(r, ZROWS)],
                            out_hbm.at[c, h, pl.ds(r, ZROWS)])


_spmm = pl.kernel(
    _spmm_body,
    out_type=jax.ShapeDtypeStruct((NC, 2, NP, DH), jnp.float32),
    mesh=plsc.VectorSubcoreMesh(core_axis_name="c", subcore_axis_name="s",
                                num_cores=NC, num_subcores=NS),
    scratch_types=[
        pltpu.VMEM((NCHUNK, K), jnp.int32),
        pltpu.VMEM((2, NCHUNK, K), jnp.int32),
        pltpu.VMEM((NCHUNK, K), jnp.float32),
        pltpu.VMEM((2, K, DH), jnp.float32),
        pltpu.VMEM((ZROWS, DH), jnp.float32),
        pltpu.VMEM_SHARED((NP, DH), jnp.float32),
        pltpu.SemaphoreType.DMA,
        pltpu.SemaphoreType.DMA,
    ],
    compiler_params=pltpu.CompilerParams(use_tc_tiling_on_sc=False),
)


def _mm_body(y_ref, w_ref, o_ref):
    o_ref[...] = (
        jnp.dot(y_ref[0, 0] + y_ref[1, 0], w_ref[0],
                preferred_element_type=jnp.float32)
        + jnp.dot(y_ref[0, 1] + y_ref[1, 1], w_ref[1],
                  preferred_element_type=jnp.float32)
    )


BM = 1000


def _matmul(y, w2):
    return pl.pallas_call(
        _mm_body,
        grid=(N // BM,),
        in_specs=[
            pl.BlockSpec((NC, 2, BM, DH), lambda i: (0, 0, i, 0)),
            pl.BlockSpec((2, DH, D), lambda i: (0, 0, 0)),
        ],
        out_specs=pl.BlockSpec((BM, D), lambda i: (i, 0)),
        out_shape=jax.ShapeDtypeStruct((N, D), jnp.float32),
    )(y, w2)


def kernel(adj_indices, adj_values, embeds, W):
    rows = adj_indices[0].astype(jnp.int32).reshape(NW, NCHUNK, K)
    c2 = (adj_indices[1].astype(jnp.int32) * 2).reshape(NW, 1, NCHUNK, K)
    cols2 = jnp.concatenate([c2, c2 + 1], axis=1)
    vals = adj_values.reshape(NW, NCHUNK, K)
    x2 = embeds.reshape(2 * N, DH)
    y = _spmm(rows, cols2, vals, x2)
    return _matmul(y[:, :, :N], W.reshape(2, DH, D))


# speedup vs baseline: 7.6083x; 1.8248x over previous
import jax
import jax.numpy as jnp
from jax import lax
from jax.experimental import pallas as pl
from jax.experimental.pallas import tpu as pltpu
from jax.experimental.pallas import tpu_sc as plsc

N = 10000
E = 320000
D = 128
DH = D // 2
L = 16
NC = 2
NS = 16
NW = NC * NS
EPT = E // NW
K = 80
NCHUNK = EPT // K
NP = 10240
RPT = NP // NS
ZROWS = 128


def _spmm_body(rows_hbm, cols2_hbm, vals_hbm, x2_hbm, out_hbm,
               rows_v, cols2_v, vals_v, gbuf, sbuf, zbuf, acc, gsem0, gsem1):
    c = lax.axis_index("c")
    s = lax.axis_index("s")
    wid = s * NC + c
    row0 = s * RPT

    pltpu.sync_copy(rows_hbm.at[wid], rows_v)
    pltpu.sync_copy(cols2_hbm.at[wid], cols2_v)
    pltpu.sync_copy(vals_hbm.at[wid], vals_v)

    def zfill(i, carry):
        for j in range(DH // L):
            zbuf[i, pl.ds(j * L, L)] = jnp.zeros((L,), jnp.float32)
        return carry
    lax.fori_loop(0, ZROWS, zfill, 0)

    sems = (gsem0, gsem1)

    for h in range(2):
        for i in range(RPT // ZROWS):
            pltpu.sync_copy(zbuf, acc.at[pl.ds(row0 + i * ZROWS, ZROWS)])
        plsc.subcore_barrier()

        def scale_scatter(ci, b):
            pltpu.make_async_copy(
                x2_hbm.at[cols2_v.at[h, 0]], gbuf.at[b], sems[b]).wait()

            for g in range(K // L):
                vv16 = vals_v[ci, pl.ds(g * L, L)]
                for e in range(L):
                    vv = jnp.take_along_axis(
                        vv16, jnp.full((L,), e, jnp.int32), axis=0)
                    row = g * L + e
                    for j in range(DH // L):
                        sl = pl.ds(j * L, L)
                        sbuf[b, row, sl] = gbuf[b, row, sl] * vv

            pltpu.sync_copy(sbuf.at[b], acc.at[rows_v.at[ci]], add=True)

        pltpu.async_copy(x2_hbm.at[cols2_v.at[h, 0]], gbuf.at[0], gsem0)

        def pair_body(p, carry):
            ci = 2 * p
            pltpu.async_copy(
                x2_hbm.at[cols2_v.at[h, ci + 1]], gbuf.at[1], gsem1)
            scale_scatter(ci, 0)
            pltpu.async_copy(
                x2_hbm.at[cols2_v.at[h, ci + 2]], gbuf.at[0], gsem0)
            scale_scatter(ci + 1, 1)
            return carry
        lax.fori_loop(0, (NCHUNK - 1) // 2, pair_body, 0)
        scale_scatter(NCHUNK - 1, 0)

        plsc.subcore_barrier()
        for i in range(RPT // ZROWS):
            r = row0 + i * ZROWS
            pltpu.sync_copy(acc.at[pl.ds(r, ZROWS)],
                            out_hbm.at[c, h, pl.ds(r, ZROWS)])


_spmm = pl.kernel(
    _spmm_body,
    out_type=jax.ShapeDtypeStruct((NC, 2, NP, DH), jnp.float32),
    mesh=plsc.VectorSubcoreMesh(core_axis_name="c", subcore_axis_name="s",
                                num_cores=NC, num_subcores=NS),
    scratch_types=[
        pltpu.VMEM((NCHUNK, K), jnp.int32),
        pltpu.VMEM((2, NCHUNK, K), jnp.int32),
        pltpu.VMEM((NCHUNK, K), jnp.float32),
        pltpu.VMEM((2, K, DH), jnp.float32),
        pltpu.VMEM((2, K, DH), jnp.float32),
        pltpu.VMEM((ZROWS, DH), jnp.float32),
        pltpu.VMEM_SHARED((NP, DH), jnp.float32),
        pltpu.SemaphoreType.DMA,
        pltpu.SemaphoreType.DMA,
    ],
    compiler_params=pltpu.CompilerParams(use_tc_tiling_on_sc=False),
)


def _mm_body(y_ref, w_ref, o_ref):
    o_ref[...] = (
        jnp.dot(y_ref[0, 0] + y_ref[1, 0], w_ref[0],
                preferred_element_type=jnp.float32)
        + jnp.dot(y_ref[0, 1] + y_ref[1, 1], w_ref[1],
                  preferred_element_type=jnp.float32)
    )


BM = 1000


def _matmul(y, w2):
    return pl.pallas_call(
        _mm_body,
        grid=(N // BM,),
        in_specs=[
            pl.BlockSpec((NC, 2, BM, DH), lambda i: (0, 0, i, 0)),
            pl.BlockSpec((2, DH, D), lambda i: (0, 0, 0)),
        ],
        out_specs=pl.BlockSpec((BM, D), lambda i: (i, 0)),
        out_shape=jax.ShapeDtypeStruct((N, D), jnp.float32),
    )(y, w2)


def kernel(adj_indices, adj_values, embeds, W):
    rows = adj_indices[0].astype(jnp.int32).reshape(NW, NCHUNK, K)
    c2 = (adj_indices[1].astype(jnp.int32) * 2).reshape(NW, 1, NCHUNK, K)
    cols2 = jnp.concatenate([c2, c2 + 1], axis=1)
    vals = adj_values.reshape(NW, NCHUNK, K)
    x2 = embeds.reshape(2 * N, DH)
    y = _spmm(rows, cols2, vals, x2)
    return _matmul(y[:, :, :N], W.reshape(2, DH, D))


# speedup vs baseline: 9.4255x; 1.2388x over previous
import jax
import jax.numpy as jnp
from jax import lax
from jax.experimental import pallas as pl
from jax.experimental.pallas import tpu as pltpu
from jax.experimental.pallas import tpu_sc as plsc

N = 10000
E = 320000
D = 128
DH = D // 2
L = 16
NC = 2
NS = 16
NW = NC * NS
EPT = E // NW
K = 80
NCHUNK = EPT // K
NP = 10240
RPT = NP // NS
ZROWS = 128


def _spmm_body(rows_hbm, cols_hbm, vals_hbm, x2_hbm, out_hbm,
               rows_v, cols_v, vals_v, cidx_v, gbuf, sbuf, zbuf, acc,
               gsem0, gsem1, ssem0, ssem1):
    c = lax.axis_index("c")
    s = lax.axis_index("s")
    wid = s * NC + c
    row0 = s * RPT

    pltpu.sync_copy(rows_hbm.at[wid], rows_v)
    pltpu.sync_copy(cols_hbm.at[wid], cols_v)
    pltpu.sync_copy(vals_hbm.at[wid], vals_v)

    def zfill(i, carry):
        for j in range(DH // L):
            zbuf[i, pl.ds(j * L, L)] = jnp.zeros((L,), jnp.float32)
        return carry
    lax.fori_loop(0, ZROWS, zfill, 0)

    gsems = (gsem0, gsem1)
    ssems = (ssem0, ssem1)

    for h in range(2):
        for i in range(RPT // ZROWS):
            pltpu.sync_copy(zbuf, acc.at[pl.ds(row0 + i * ZROWS, ZROWS)])
        plsc.subcore_barrier()

        def start_gather(ci, b):
            for g in range(K // L):
                sl = pl.ds(g * L, L)
                cidx_v[b, sl] = cols_v[ci, sl] * 2 + h
            pltpu.async_copy(x2_hbm.at[cidx_v.at[b]], gbuf.at[b], gsems[b])

        def scale_scatter(ci, b):
            pltpu.make_async_copy(
                x2_hbm.at[cidx_v.at[b]], gbuf.at[b], gsems[b]).wait()
            pltpu.make_async_copy(
                sbuf.at[b], acc.at[rows_v.at[0]], ssems[b]).wait()

            for g in range(K // L):
                vv16 = vals_v[ci, pl.ds(g * L, L)]
                for e in range(L):
                    vv = jnp.take_along_axis(
                        vv16, jnp.full((L,), e, jnp.int32), axis=0)
                    row = g * L + e
                    for j in range(DH // L):
                        sl = pl.ds(j * L, L)
                        sbuf[b, row, sl] = gbuf[b, row, sl] * vv

            pltpu.async_copy(
                sbuf.at[b], acc.at[rows_v.at[ci]], ssems[b], add=True)

        pltpu.async_copy(
            zbuf.at[pl.ds(0, K)], acc.at[rows_v.at[0]], ssem0, add=True)
        pltpu.async_copy(
            zbuf.at[pl.ds(0, K)], acc.at[rows_v.at[0]], ssem1, add=True)

        start_gather(0, 0)

        def pair_body(p, carry):
            ci = 2 * p
            start_gather(ci + 1, 1)
            scale_scatter(ci, 0)
            start_gather(ci + 2, 0)
            scale_scatter(ci + 1, 1)
            return carry
        lax.fori_loop(0, (NCHUNK - 1) // 2, pair_body, 0)
        scale_scatter(NCHUNK - 1, 0)

        pltpu.make_async_copy(
            sbuf.at[0], acc.at[rows_v.at[0]], ssem0).wait()
        pltpu.make_async_copy(
            sbuf.at[1], acc.at[rows_v.at[0]], ssem1).wait()

        plsc.subcore_barrier()
        for i in range(RPT // ZROWS):
            r = row0 + i * ZROWS
            pltpu.sync_copy(acc.at[pl.ds(r, ZROWS)],
                            out_hbm.at[c, h, pl.ds(r, ZROWS)])


_spmm = pl.kernel(
    _spmm_body,
    out_type=jax.ShapeDtypeStruct((NC, 2, NP, DH), jnp.float32),
    mesh=plsc.VectorSubcoreMesh(core_axis_name="c", subcore_axis_name="s",
                                num_cores=NC, num_subcores=NS),
    scratch_types=[
        pltpu.VMEM((NCHUNK, K), jnp.int32),
        pltpu.VMEM((NCHUNK, K), jnp.int32),
        pltpu.VMEM((NCHUNK, K), jnp.float32),
        pltpu.VMEM((2, K), jnp.int32),
        pltpu.VMEM((2, K, DH), jnp.float32),
        pltpu.VMEM((2, K, DH), jnp.float32),
        pltpu.VMEM((ZROWS, DH), jnp.float32),
        pltpu.VMEM_SHARED((NP, DH), jnp.float32),
        pltpu.SemaphoreType.DMA,
        pltpu.SemaphoreType.DMA,
        pltpu.SemaphoreType.DMA,
        pltpu.SemaphoreType.DMA,
    ],
    compiler_params=pltpu.CompilerParams(use_tc_tiling_on_sc=False),
)


def _mm_body(y_ref, w_ref, o_ref):
    o_ref[...] = (
        jnp.dot(y_ref[0, 0] + y_ref[1, 0], w_ref[0],
                preferred_element_type=jnp.float32)
        + jnp.dot(y_ref[0, 1] + y_ref[1, 1], w_ref[1],
                  preferred_element_type=jnp.float32)
    )


BM = 1000


def _matmul(y, w2):
    return pl.pallas_call(
        _mm_body,
        grid=(N // BM,),
        in_specs=[
            pl.BlockSpec((NC, 2, BM, DH), lambda i: (0, 0, i, 0)),
            pl.BlockSpec((2, DH, D), lambda i: (0, 0, 0)),
        ],
        out_specs=pl.BlockSpec((BM, D), lambda i: (i, 0)),
        out_shape=jax.ShapeDtypeStruct((N, D), jnp.float32),
    )(y, w2)


def kernel(adj_indices, adj_values, embeds, W):
    rows = adj_indices[0].astype(jnp.int32).reshape(NW, NCHUNK, K)
    cols = adj_indices[1].astype(jnp.int32).reshape(NW, NCHUNK, K)
    vals = adj_values.reshape(NW, NCHUNK, K)
    x2 = embeds.reshape(2 * N, DH)
    y = _spmm(rows, cols, vals, x2)
    return _matmul(y, W.reshape(2, DH, D))


# speedup vs baseline: 11.6157x; 1.2324x over previous
import jax
import jax.numpy as jnp
from jax import lax
from jax.experimental import pallas as pl
from jax.experimental.pallas import tpu as pltpu
from jax.experimental.pallas import tpu_sc as plsc

N = 10000
E = 320000
D = 128
DH = D // 2
L = 16
NC = 2
NS = 16
NW = NC * NS
EPT = E // NW
K = 80
NCHUNK = EPT // K
NP = 10240
RPT = NP // NS
ZROWS = 128


NB = 4


def _spmm_body(adj_hbm, vals_hbm, x2_hbm, out_hbm,
               rows_v, cols_v, vals_v, cidx_v, gbuf, sbuf, zbuf, acc,
               gsem0, gsem1, gsem2, gsem3, ssem0, ssem1, ssem2, ssem3):
    c = lax.axis_index("c")
    s = lax.axis_index("s")
    wid = s * NC + c
    row0 = s * RPT

    pltpu.sync_copy(adj_hbm.at[0, wid], rows_v)
    pltpu.sync_copy(adj_hbm.at[1, wid], cols_v)
    pltpu.sync_copy(vals_hbm.at[wid], vals_v)

    def zfill(i, carry):
        for j in range(DH // L):
            zbuf[i, pl.ds(j * L, L)] = jnp.zeros((L,), jnp.float32)
        return carry
    lax.fori_loop(0, ZROWS, zfill, 0)

    gsems = (gsem0, gsem1, gsem2, gsem3)
    ssems = (ssem0, ssem1, ssem2, ssem3)

    for h in range(2):
        for i in range(RPT // ZROWS):
            pltpu.sync_copy(zbuf, acc.at[pl.ds(row0 + i * ZROWS, ZROWS)])
        plsc.subcore_barrier()

        def start_gather(ci, b):
            for g in range(K // L):
                sl = pl.ds(g * L, L)
                cidx_v[b, sl] = cols_v[ci, sl] * 2 + h
            pltpu.async_copy(x2_hbm.at[cidx_v.at[b]], gbuf.at[b], gsems[b])

        def scale_scatter(ci, b, first=False):
            pltpu.make_async_copy(
                x2_hbm.at[cidx_v.at[b]], gbuf.at[b], gsems[b]).wait()
            if not first:
                pltpu.make_async_copy(
                    sbuf.at[b], acc.at[rows_v.at[0]], ssems[b]).wait()

            def group_body(g, inner):
                vv16 = vals_v[ci, pl.ds(g * L, L)]
                for e in range(L):
                    vv = jnp.take_along_axis(
                        vv16, jnp.full((L,), e, jnp.int32), axis=0)
                    for j in range(DH // L):
                        sl = pl.ds(j * L, L)
                        row = g * L + e
                        sbuf[b, row, sl] = gbuf[b, row, sl] * vv
                return inner
            lax.fori_loop(0, K // L, group_body, 0)

            pltpu.async_copy(
                sbuf.at[b], acc.at[rows_v.at[ci]], ssems[b], add=True)

        for b in range(NB - 1):
            start_gather(b, b)

        for q in range(NB):
            ci = q
            start_gather(ci + NB - 1, (ci + NB - 1) % NB)
            scale_scatter(ci, ci % NB, first=True)

        def quad_body(p, carry):
            ci0 = NB * p
            for q in range(NB):
                ci = ci0 + q
                start_gather(ci + NB - 1, (q + NB - 1) % NB)
                scale_scatter(ci, q)
            return carry
        lax.fori_loop(1, (NCHUNK - 1 - NB) // NB, quad_body, 0)

        for ci in range((((NCHUNK - 1 - NB) // NB) - 1) * NB + NB, NCHUNK):
            if ci + NB - 1 < NCHUNK:
                start_gather(ci + NB - 1, (ci + NB - 1) % NB)
            scale_scatter(ci, ci % NB)

        for b in range(NB):
            pltpu.make_async_copy(
                sbuf.at[b], acc.at[rows_v.at[0]], ssems[b]).wait()

        plsc.subcore_barrier()
        for i in range(RPT // ZROWS):
            r = row0 + i * ZROWS
            pltpu.sync_copy(acc.at[pl.ds(r, ZROWS)],
                            out_hbm.at[c, h, pl.ds(r, ZROWS)])


_spmm = pl.kernel(
    _spmm_body,
    out_type=jax.ShapeDtypeStruct((NC, 2, NP, DH), jnp.float32),
    mesh=plsc.VectorSubcoreMesh(core_axis_name="c", subcore_axis_name="s",
                                num_cores=NC, num_subcores=NS),
    scratch_types=[
        pltpu.VMEM((NCHUNK, K), jnp.int32),
        pltpu.VMEM((NCHUNK, K), jnp.int32),
        pltpu.VMEM((NCHUNK, K), jnp.float32),
        pltpu.VMEM((NB, K), jnp.int32),
        pltpu.VMEM((NB, K, DH), jnp.float32),
        pltpu.VMEM((NB, K, DH), jnp.float32),
        pltpu.VMEM((ZROWS, DH), jnp.float32),
        pltpu.VMEM_SHARED((NP, DH), jnp.float32),
        pltpu.SemaphoreType.DMA,
        pltpu.SemaphoreType.DMA,
        pltpu.SemaphoreType.DMA,
        pltpu.SemaphoreType.DMA,
        pltpu.SemaphoreType.DMA,
        pltpu.SemaphoreType.DMA,
        pltpu.SemaphoreType.DMA,
        pltpu.SemaphoreType.DMA,
    ],
    compiler_params=pltpu.CompilerParams(use_tc_tiling_on_sc=False),
)


def _mm_body(y_ref, w_ref, o_ref):
    o_ref[...] = (
        jnp.dot(y_ref[0, 0] + y_ref[1, 0], w_ref[0],
                preferred_element_type=jnp.float32)
        + jnp.dot(y_ref[0, 1] + y_ref[1, 1], w_ref[1],
                  preferred_element_type=jnp.float32)
    )


BM = 1000


def _matmul(y, w2):
    return pl.pallas_call(
        _mm_body,
        grid=(N // BM,),
        in_specs=[
            pl.BlockSpec((NC, 2, BM, DH), lambda i: (0, 0, i, 0)),
            pl.BlockSpec((2, DH, D), lambda i: (0, 0, 0)),
        ],
        out_specs=pl.BlockSpec((BM, D), lambda i: (i, 0)),
        out_shape=jax.ShapeDtypeStruct((N, D), jnp.float32),
    )(y, w2)


def kernel(adj_indices, adj_values, embeds, W):
    adj = adj_indices.astype(jnp.int32).reshape(2, NW, NCHUNK, K)
    vals = adj_values.reshape(NW, NCHUNK, K)
    x2 = embeds.reshape(2 * N, DH)
    y = _spmm(adj, vals, x2)
    return _matmul(y, W.reshape(2, DH, D))
